# baseline (speedup 1.0000x reference)
import functools

import jax
import jax.numpy as jnp
from jax import lax
from jax.experimental import pallas as pl
from jax.experimental.pallas import tpu as pltpu
from jax.experimental.pallas import tpu_sc as plsc

T = 2048
D = 768
DFF = 3072
E = 8
K = 2
BT = 256
NB = (T * K) // BT + (E - 1)
G = NB * BT
BF = 768
NF = DFF // BF
NW = 32
TPW = T // NW


def _route_body(x_ref, wg_ref, bg_ref, pos_ref, cb_ref):
    x = x_ref[...]
    logits = jnp.dot(x, wg_ref[...], preferred_element_type=jnp.float32)
    logits = logits + bg_ref[...]
    lane = lax.broadcasted_iota(jnp.int32, (T, E), 1)
    m0 = jnp.max(logits, axis=1, keepdims=True)
    i0 = jnp.min(jnp.where(logits == m0, lane, E), axis=1, keepdims=True)
    oh0 = (lane == i0)
    l2 = jnp.where(oh0, -jnp.inf, logits)
    m1 = jnp.max(l2, axis=1, keepdims=True)
    i1 = jnp.min(jnp.where(l2 == m1, lane, E), axis=1, keepdims=True)
    oh1 = (lane == i1)

    oh = jnp.concatenate([oh0.astype(jnp.float32), oh1.astype(jnp.float32)],
                         axis=0)
    r = lax.broadcasted_iota(jnp.int32, (BT, BT), 0)
    c = lax.broadcasted_iota(jnp.int32, (BT, BT), 1)
    tri = (r >= c).astype(jnp.float32)
    nchunks = (T * K) // BT
    carry = jnp.zeros((1, E), jnp.float32)
    pieces = []
    for j in range(nchunks):
        w = jnp.dot(tri, oh[j * BT:(j + 1) * BT, :],
                    preferred_element_type=jnp.float32) + carry
        pieces.append(w)
        carry = w[BT - 1:BT, :]
    incl = jnp.concatenate(pieces, axis=0)
    n = carry
    nblk = (n.astype(jnp.int32) + (BT - 1)) >> 8
    er = lax.broadcasted_iota(jnp.int32, (E, E), 0)
    ec = lax.broadcasted_iota(jnp.int32, (E, E), 1)
    triu = (er <= ec).astype(jnp.float32)
    cbf = jnp.dot(nblk.astype(jnp.float32), triu,
                  preferred_element_type=jnp.float32)
    start = (cbf - nblk.astype(jnp.float32)) * float(BT)
    pos = jnp.sum(oh * (incl + start), axis=1, keepdims=True) - 1.0
    pos_ref[...] = pos.astype(jnp.int32)

    cbi = cbf.astype(jnp.int32)
    cbx = cbi - nblk
    cb_ref[...] = jnp.concatenate([cbx, cbi], axis=0)


def _route(x, Wg, bg):
    return pl.pallas_call(
        _route_body,
        out_shape=(
            jax.ShapeDtypeStruct((T * K, 1), jnp.int32),
            jax.ShapeDtypeStruct((2, E), jnp.int32),
        ),
    )(x, Wg, bg.reshape(1, E))


def _dispatch_body(x_hbm, p0_hbm, p1_hbm, xg_hbm, i0_v, i1_v, xb_v, s0, s1):
    wid = lax.axis_index("s") * 2 + lax.axis_index("c")
    base = wid * TPW
    pltpu.sync_copy(p0_hbm.at[pl.ds(base, TPW)], i0_v)
    pltpu.sync_copy(p1_hbm.at[pl.ds(base, TPW)], i1_v)
    pltpu.sync_copy(x_hbm.at[pl.ds(base, TPW)], xb_v)
    c0 = pltpu.async_copy(xb_v, xg_hbm.at[i0_v], s0)
    c1 = pltpu.async_copy(xb_v, xg_hbm.at[i1_v], s1)
    c0.wait()
    c1.wait()


def _dispatch(x, p0, p1):
    return pl.kernel(
        _dispatch_body,
        out_type=jax.ShapeDtypeStruct((G, D), jnp.float32),
        mesh=plsc.VectorSubcoreMesh(core_axis_name="c", subcore_axis_name="s"),
        scratch_types=[
            pltpu.VMEM((TPW,), jnp.int32),
            pltpu.VMEM((TPW,), jnp.int32),
            pltpu.VMEM((TPW, D), jnp.float32),
            pltpu.SemaphoreType.DMA,
            pltpu.SemaphoreType.DMA,
        ],
    )(x, p0, p1)


def _w_copies(w1_hbm, w2_hbm, w1b, w2b, sems, sn):
    en = sn // NF
    fn = sn % NF
    slot = sn % 2
    c1 = pltpu.make_async_copy(w1_hbm.at[en, :, pl.ds(fn * BF, BF)],
                               w1b.at[slot], sems.at[slot, 0])
    c2 = pltpu.make_async_copy(w2_hbm.at[en, pl.ds(fn * BF, BF), :],
                               w2b.at[slot], sems.at[slot, 1])
    return c1, c2


def _experts_body(cb_ref, xg_ref, w1_hbm, b1_ref, w2_hbm, b2_ref, y_ref,
                  w1b, w2b, sems):
    e = pl.program_id(0)
    f = pl.program_id(1)
    s = e * NF + f
    lo = cb_ref[0, e]
    hi = cb_ref[1, e]

    @pl.when(s == 0)
    def _():
        c1, c2 = _w_copies(w1_hbm, w2_hbm, w1b, w2b, sems, 0)
        c1.start()
        c2.start()

    @pl.when(s + 1 < E * NF)
    def _():
        c1, c2 = _w_copies(w1_hbm, w2_hbm, w1b, w2b, sems, s + 1)
        c1.start()
        c2.start()

    c1, c2 = _w_copies(w1_hbm, w2_hbm, w1b, w2b, sems, s)
    c1.wait()
    c2.wait()

    slot = s % 2
    w1 = w1b[slot].astype(jnp.bfloat16)
    w2 = w2b[slot].astype(jnp.bfloat16)
    b1v = b1_ref[pl.ds(e, 1), pl.ds(f * BF, BF)]
    b2v = b2_ref[pl.ds(e, 1), :]

    def one(b):
        rows = pl.ds(b * BT, BT)
        xb = xg_ref[rows, :].astype(jnp.bfloat16)
        h = jnp.dot(xb, w1, preferred_element_type=jnp.float32) + b1v
        h = h * jax.nn.sigmoid(h)
        part = jnp.dot(h.astype(jnp.bfloat16), w2,
                       preferred_element_type=jnp.float32)

        @pl.when(f == 0)
        def _():
            y_ref[rows, :] = part + b2v

        @pl.when(f > 0)
        def _():
            y_ref[rows, :] += part

    def blk(b, carry):
        one(b)
        return carry

    lax.fori_loop(lo, hi, blk, 0)


def _experts(cb, xg, W1, b1, W2, b2):
    grid_spec = pltpu.PrefetchScalarGridSpec(
        num_scalar_prefetch=1,
        grid=(E, NF),
        in_specs=[
            pl.BlockSpec((G, D), lambda e, f, cb: (0, 0)),
            pl.BlockSpec(memory_space=pl.ANY),
            pl.BlockSpec((E, DFF), lambda e, f, cb: (0, 0)),
            pl.BlockSpec(memory_space=pl.ANY),
            pl.BlockSpec((E, D), lambda e, f, cb: (0, 0)),
        ],
        out_specs=pl.BlockSpec((G, D), lambda e, f, cb: (0, 0)),
        scratch_shapes=[
            pltpu.VMEM((2, D, BF), jnp.float32),
            pltpu.VMEM((2, BF, D), jnp.float32),
            pltpu.SemaphoreType.DMA((2, 2)),
        ],
    )
    return pl.pallas_call(
        _experts_body,
        grid_spec=grid_spec,
        out_shape=jax.ShapeDtypeStruct((G, D), jnp.float32),
        compiler_params=pltpu.CompilerParams(
            dimension_semantics=("arbitrary", "arbitrary")),
    )(cb, xg, W1, b1, W2, b2)


def _combine_body(y_hbm, p0_hbm, p1_hbm, out_hbm, i0_v, i1_v, g0_v, g1_v,
                  s0, s1):
    wid = lax.axis_index("s") * 2 + lax.axis_index("c")
    base = wid * TPW
    pltpu.sync_copy(p0_hbm.at[pl.ds(base, TPW)], i0_v)
    pltpu.sync_copy(p1_hbm.at[pl.ds(base, TPW)], i1_v)
    c0 = pltpu.async_copy(y_hbm.at[i0_v], g0_v, s0)
    c1 = pltpu.async_copy(y_hbm.at[i1_v], g1_v, s1)
    c0.wait()
    c1.wait()

    def row(rr, _):
        for cc in range(D // 16):
            sl = pl.ds(cc * 16, 16)
            g0_v[rr, sl] = g0_v[rr, sl] + g1_v[rr, sl]
        return _

    lax.fori_loop(0, TPW, row, 0)
    pltpu.sync_copy(g0_v, out_hbm.at[pl.ds(base, TPW)])


def _combine(y, p0, p1):
    return pl.kernel(
        _combine_body,
        out_type=jax.ShapeDtypeStruct((T, D), jnp.float32),
        mesh=plsc.VectorSubcoreMesh(core_axis_name="c", subcore_axis_name="s"),
        scratch_types=[
            pltpu.VMEM((TPW,), jnp.int32),
            pltpu.VMEM((TPW,), jnp.int32),
            pltpu.VMEM((TPW, D), jnp.float32),
            pltpu.VMEM((TPW, D), jnp.float32),
            pltpu.SemaphoreType.DMA,
            pltpu.SemaphoreType.DMA,
        ],
    )(y, p0, p1)


def kernel(x, Wg, bg, W1, b1, W2, b2):
    pos, cb = _route(x, Wg, bg)
    p0 = pos[:T, 0]
    p1 = pos[T:, 0]
    xg = _dispatch(x, p0, p1)
    y = _experts(cb, xg, W1, b1, W2, b2)
    return _combine(y, p0, p1)


# speedup vs baseline: 1.1037x; 1.1037x over previous
import functools

import jax
import jax.numpy as jnp
from jax import lax
from jax.experimental import pallas as pl
from jax.experimental.pallas import tpu as pltpu
from jax.experimental.pallas import tpu_sc as plsc

T = 2048
D = 768
DFF = 3072
E = 8
K = 2
BT = 256
NB = (T * K) // BT + (E - 1)
G = NB * BT
BF = 1024
NF = DFF // BF
NW = 32
TPW = T // NW


def _route_body(x_ref, wg_ref, bg_ref, pos_ref, cb_ref):
    x = x_ref[...]
    logits = jnp.dot(x, wg_ref[...], preferred_element_type=jnp.float32)
    logits = logits + bg_ref[...]
    lane = lax.broadcasted_iota(jnp.int32, (T, E), 1)
    m0 = jnp.max(logits, axis=1, keepdims=True)
    i0 = jnp.min(jnp.where(logits == m0, lane, E), axis=1, keepdims=True)
    oh0 = (lane == i0)
    l2 = jnp.where(oh0, -jnp.inf, logits)
    m1 = jnp.max(l2, axis=1, keepdims=True)
    i1 = jnp.min(jnp.where(l2 == m1, lane, E), axis=1, keepdims=True)
    oh1 = (lane == i1)

    oh = jnp.concatenate([oh0.astype(jnp.float32), oh1.astype(jnp.float32)],
                         axis=0)
    r = lax.broadcasted_iota(jnp.int32, (BT, BT), 0)
    c = lax.broadcasted_iota(jnp.int32, (BT, BT), 1)
    tri = (r >= c).astype(jnp.float32)
    nchunks = (T * K) // BT
    carry = jnp.zeros((1, E), jnp.float32)
    pieces = []
    for j in range(nchunks):
        w = jnp.dot(tri, oh[j * BT:(j + 1) * BT, :],
                    preferred_element_type=jnp.float32) + carry
        pieces.append(w)
        carry = w[BT - 1:BT, :]
    incl = jnp.concatenate(pieces, axis=0)
    n = carry
    nblk = (n.astype(jnp.int32) + (BT - 1)) >> 8
    er = lax.broadcasted_iota(jnp.int32, (E, E), 0)
    ec = lax.broadcasted_iota(jnp.int32, (E, E), 1)
    triu = (er <= ec).astype(jnp.float32)
    cbf = jnp.dot(nblk.astype(jnp.float32), triu,
                  preferred_element_type=jnp.float32)
    start = (cbf - nblk.astype(jnp.float32)) * float(BT)
    pos = jnp.sum(oh * (incl + start), axis=1, keepdims=True) - 1.0
    pos_ref[...] = pos.astype(jnp.int32)

    cbi = cbf.astype(jnp.int32)
    cbx = cbi - nblk
    cb_ref[...] = jnp.concatenate([cbx, cbi], axis=0)


def _route(x, Wg, bg):
    return pl.pallas_call(
        _route_body,
        out_shape=(
            jax.ShapeDtypeStruct((T * K, 1), jnp.int32),
            jax.ShapeDtypeStruct((2, E), jnp.int32),
        ),
    )(x, Wg, bg.reshape(1, E))


def _dispatch_body(x_hbm, p0_hbm, p1_hbm, xg_hbm, i0_v, i1_v, xb_v, s0, s1):
    wid = lax.axis_index("s") * 2 + lax.axis_index("c")
    base = wid * TPW
    pltpu.sync_copy(p0_hbm.at[pl.ds(base, TPW)], i0_v)
    pltpu.sync_copy(p1_hbm.at[pl.ds(base, TPW)], i1_v)
    pltpu.sync_copy(x_hbm.at[pl.ds(base, TPW)], xb_v)
    c0 = pltpu.async_copy(xb_v, xg_hbm.at[i0_v], s0)
    c1 = pltpu.async_copy(xb_v, xg_hbm.at[i1_v], s1)
    c0.wait()
    c1.wait()


def _dispatch(x, p0, p1):
    return pl.kernel(
        _dispatch_body,
        out_type=jax.ShapeDtypeStruct((G, D), jnp.float32),
        mesh=plsc.VectorSubcoreMesh(core_axis_name="c", subcore_axis_name="s"),
        scratch_types=[
            pltpu.VMEM((TPW,), jnp.int32),
            pltpu.VMEM((TPW,), jnp.int32),
            pltpu.VMEM((TPW, D), jnp.float32),
            pltpu.SemaphoreType.DMA,
            pltpu.SemaphoreType.DMA,
        ],
    )(x, p0, p1)


def _experts_body(cb_ref, xg_ref, w1_ref, b1_ref, w2_ref, b2_ref, y_ref):
    e = pl.program_id(0)
    f = pl.program_id(1)
    lo = cb_ref[0, e]
    hi = cb_ref[1, e]
    w1 = w1_ref[0].astype(jnp.bfloat16)
    w2 = w2_ref[0].astype(jnp.bfloat16)
    b1v = b1_ref[0, 0]
    b2v = b2_ref[0]

    def one(b):
        rows = pl.ds(b * BT, BT)
        xb = xg_ref[rows, :].astype(jnp.bfloat16)
        h = jnp.dot(xb, w1, preferred_element_type=jnp.float32) + b1v
        h = h * jax.nn.sigmoid(h)
        part = jnp.dot(h.astype(jnp.bfloat16), w2,
                       preferred_element_type=jnp.float32)

        @pl.when(f == 0)
        def _():
            y_ref[rows, :] = part + b2v

        @pl.when(f > 0)
        def _():
            y_ref[rows, :] += part

    def pair(i, carry):
        b = lo + 2 * i
        one(b)
        one(b + 1)
        return carry

    n = hi - lo
    lax.fori_loop(0, n // 2, pair, 0)

    @pl.when(n % 2 == 1)
    def _():
        one(hi - 1)


def _experts(cb, xg, W1, b1, W2, b2):
    grid_spec = pltpu.PrefetchScalarGridSpec(
        num_scalar_prefetch=1,
        grid=(E, NF),
        in_specs=[
            pl.BlockSpec((G, D), lambda e, f, cb: (0, 0)),
            pl.BlockSpec((1, D, BF), lambda e, f, cb: (e, 0, f)),
            pl.BlockSpec((1, 1, 1, BF), lambda e, f, cb: (e, f, 0, 0)),
            pl.BlockSpec((1, BF, D), lambda e, f, cb: (e, f, 0)),
            pl.BlockSpec((1, 1, D), lambda e, f, cb: (e, 0, 0)),
        ],
        out_specs=pl.BlockSpec((G, D), lambda e, f, cb: (0, 0)),
    )
    return pl.pallas_call(
        _experts_body,
        grid_spec=grid_spec,
        out_shape=jax.ShapeDtypeStruct((G, D), jnp.float32),
        compiler_params=pltpu.CompilerParams(
            dimension_semantics=("arbitrary", "arbitrary")),
    )(cb, xg, W1, b1.reshape(E, NF, 1, BF), W2, b2.reshape(E, 1, D))


def _combine_body(y_hbm, p0_hbm, p1_hbm, out_hbm, i0_v, i1_v, g0_v, g1_v,
                  s0, s1):
    wid = lax.axis_index("s") * 2 + lax.axis_index("c")
    base = wid * TPW
    pltpu.sync_copy(p0_hbm.at[pl.ds(base, TPW)], i0_v)
    pltpu.sync_copy(p1_hbm.at[pl.ds(base, TPW)], i1_v)
    c0 = pltpu.async_copy(y_hbm.at[i0_v], g0_v, s0)
    c1 = pltpu.async_copy(y_hbm.at[i1_v], g1_v, s1)
    c0.wait()
    c1.wait()

    def row(rr, _):
        for cc in range(D // 16):
            sl = pl.ds(cc * 16, 16)
            g0_v[rr, sl] = g0_v[rr, sl] + g1_v[rr, sl]
        return _

    lax.fori_loop(0, TPW, row, 0)
    pltpu.sync_copy(g0_v, out_hbm.at[pl.ds(base, TPW)])


def _combine(y, p0, p1):
    return pl.kernel(
        _combine_body,
        out_type=jax.ShapeDtypeStruct((T, D), jnp.float32),
        mesh=plsc.VectorSubcoreMesh(core_axis_name="c", subcore_axis_name="s"),
        scratch_types=[
            pltpu.VMEM((TPW,), jnp.int32),
            pltpu.VMEM((TPW,), jnp.int32),
            pltpu.VMEM((TPW, D), jnp.float32),
            pltpu.VMEM((TPW, D), jnp.float32),
            pltpu.SemaphoreType.DMA,
            pltpu.SemaphoreType.DMA,
        ],
    )(y, p0, p1)


def kernel(x, Wg, bg, W1, b1, W2, b2):
    pos, cb = _route(x, Wg, bg)
    p0 = pos[:T, 0]
    p1 = pos[T:, 0]
    xg = _dispatch(x, p0, p1)
    y = _experts(cb, xg, W1, b1, W2, b2)
    return _combine(y, p0, p1)
